# baseline (speedup 1.0000x reference)
import functools
import math

import jax
import jax.numpy as jnp
from jax import lax
from jax.experimental import pallas as pl
from jax.experimental.pallas import tpu as pltpu
from jax.experimental.pallas import tpu_sc as plsc

D = 512
L = 256
B = 8
VOCAB = 8000
SQRT_D = math.sqrt(float(D))

BM = 256
BN = 1024


def _sc_gather(table, idx):
    ntok = idx.shape[0]
    info = plsc.get_sparse_core_info()
    nw = info.num_cores * info.num_subcores
    b_per_w = ntok // nw

    mesh = plsc.VectorSubcoreMesh(core_axis_name="c", subcore_axis_name="s")

    @functools.partial(
        pl.kernel,
        mesh=mesh,
        out_type=jax.ShapeDtypeStruct((ntok, D), jnp.float32),
        compiler_params=pltpu.CompilerParams(use_tc_tiling_on_sc=True),
        scratch_types=[
            pltpu.VMEM((b_per_w,), jnp.int32),
            pltpu.VMEM((b_per_w, D), jnp.float32),
            pltpu.SemaphoreType.DMA,
        ],
    )
    def gather_kernel(table_hbm, idx_hbm, out_hbm, idx_v, rows_v, sem):
        wid = lax.axis_index("s") * info.num_cores + lax.axis_index("c")
        base = wid * b_per_w
        pltpu.sync_copy(idx_hbm.at[pl.ds(base, b_per_w)], idx_v)
        pltpu.async_copy(table_hbm.at[idx_v], rows_v, sem).wait()
        pltpu.sync_copy(rows_v, out_hbm.at[pl.ds(base, b_per_w)])

    return gather_kernel(table, idx)


def _mm_body(x_ref, pos_ref, w_ref, b_ref, o_ref, xbf):
    n = pl.program_id(0)
    m = pl.program_id(1)

    @pl.when(n == 0)
    def _cache_x():
        ps = pos_ref[pl.ds((m // B) * L, L), :]
        xb = x_ref[pl.ds(m * BM, BM), :] * SQRT_D + ps
        xbf[pl.ds(m * BM, BM), :] = xb.astype(jnp.bfloat16)

    acc = lax.dot_general(w_ref[...], xbf[pl.ds(m * BM, BM), :],
                          (((1,), (1,)), ((), ())),
                          preferred_element_type=jnp.float32)
    o_ref[...] = (acc + b_ref[...]).reshape(1, 1, BN, BM)


def _tc_matmul(x, pos_cat, wbf, bcT):
    ntok = x.shape[0]
    m_blocks = ntok // BM
    n_blocks = (VOCAB + BN - 1) // BN
    return pl.pallas_call(
        _mm_body,
        grid=(n_blocks, m_blocks),
        in_specs=[
            pl.BlockSpec((ntok, D), lambda n, m: (0, 0)),
            pl.BlockSpec((2 * L, D), lambda n, m: (0, 0)),
            pl.BlockSpec((BN, D), lambda n, m: (n, 0)),
            pl.BlockSpec((BN, 1), lambda n, m: (n, 0)),
        ],
        out_specs=pl.BlockSpec((1, 1, BN, BM), lambda n, m: (m // B, m % B, n, 0)),
        out_shape=jax.ShapeDtypeStruct((2, B, VOCAB, BM), jnp.float32),
        scratch_shapes=[
            pltpu.VMEM((ntok, D), jnp.bfloat16),
        ],
    )(x, pos_cat, wbf, bcT)


def kernel(src, tgt, emb, pos_src, pos_tgt, Wc, bc):
    ls = src.shape[1]
    lt = tgt.shape[1]
    idx = jnp.concatenate(
        [src.reshape(-1), tgt.reshape(-1)]).astype(jnp.int32)
    pos_cat = jnp.concatenate([pos_src[:ls, :D], pos_tgt[:lt, :D]], axis=0)
    wbf = Wc.astype(jnp.bfloat16)
    x = _sc_gather(emb, idx)
    out_t = _tc_matmul(x, pos_cat, wbf, bc.reshape(VOCAB, 1))
    return jnp.transpose(out_t, (0, 1, 3, 2))


# speedup vs baseline: 1.4581x; 1.4581x over previous
import functools
import math

import jax
import jax.numpy as jnp
from jax import lax
from jax.experimental import pallas as pl
from jax.experimental.pallas import tpu as pltpu
from jax.experimental.pallas import tpu_sc as plsc

D = 512
L = 256
B = 8
VOCAB = 8000
SQRT_D = math.sqrt(float(D))

BM = 256
BN = 4096


def _sc_gather(table, idx):
    ntok = idx.shape[0]
    info = plsc.get_sparse_core_info()
    nw = info.num_cores * info.num_subcores
    b_per_w = ntok // nw

    mesh = plsc.VectorSubcoreMesh(core_axis_name="c", subcore_axis_name="s")

    @functools.partial(
        pl.kernel,
        mesh=mesh,
        out_type=jax.ShapeDtypeStruct((ntok, D), jnp.float32),
        compiler_params=pltpu.CompilerParams(use_tc_tiling_on_sc=True),
        scratch_types=[
            pltpu.VMEM((b_per_w,), jnp.int32),
            pltpu.VMEM((b_per_w, D), jnp.float32),
            pltpu.SemaphoreType.DMA,
        ],
    )
    def gather_kernel(table_hbm, idx_hbm, out_hbm, idx_v, rows_v, sem):
        wid = lax.axis_index("s") * info.num_cores + lax.axis_index("c")
        base = wid * b_per_w
        pltpu.sync_copy(idx_hbm.at[pl.ds(base, b_per_w)], idx_v)
        pltpu.async_copy(table_hbm.at[idx_v], rows_v, sem).wait()
        pltpu.sync_copy(rows_v, out_hbm.at[pl.ds(base, b_per_w)])

    return gather_kernel(table, idx)


def _mm_body(x_ref, pos_ref, w_ref, b_ref, o_ref, xbf):
    n = pl.program_id(0)
    m = pl.program_id(1)

    @pl.when(n == 0)
    def _cache_x():
        ps = pos_ref[pl.ds((m // B) * L, L), :]
        xb = x_ref[pl.ds(m * BM, BM), :] * SQRT_D + ps
        xbf[pl.ds(m * BM, BM), :] = xb.astype(jnp.bfloat16)

    acc = lax.dot_general(w_ref[...], xbf[pl.ds(m * BM, BM), :],
                          (((1,), (1,)), ((), ())),
                          preferred_element_type=jnp.float32)
    o_ref[...] = (acc + b_ref[...]).reshape(1, 1, BN, BM)


def _tc_matmul(x, pos_cat, wbf, bcT):
    ntok = x.shape[0]
    m_blocks = ntok // BM
    n_blocks = (VOCAB + BN - 1) // BN
    return pl.pallas_call(
        _mm_body,
        grid=(n_blocks, m_blocks),
        in_specs=[
            pl.BlockSpec((ntok, D), lambda n, m: (0, 0)),
            pl.BlockSpec((2 * L, D), lambda n, m: (0, 0)),
            pl.BlockSpec((BN, D), lambda n, m: (n, 0)),
            pl.BlockSpec((BN, 1), lambda n, m: (n, 0)),
        ],
        out_specs=pl.BlockSpec((1, 1, BN, BM), lambda n, m: (m // B, m % B, n, 0)),
        out_shape=jax.ShapeDtypeStruct((2, B, VOCAB, BM), jnp.float32),
        scratch_shapes=[
            pltpu.VMEM((ntok, D), jnp.bfloat16),
        ],
    )(x, pos_cat, wbf, bcT)


def kernel(src, tgt, emb, pos_src, pos_tgt, Wc, bc):
    ls = src.shape[1]
    lt = tgt.shape[1]
    idx = jnp.concatenate(
        [src.reshape(-1), tgt.reshape(-1)]).astype(jnp.int32)
    pos_cat = jnp.concatenate([pos_src[:ls, :D], pos_tgt[:lt, :D]], axis=0)
    wbf = Wc.astype(jnp.bfloat16)
    x = _sc_gather(emb, idx)
    out_t = _tc_matmul(x, pos_cat, wbf, bc.reshape(VOCAB, 1))
    return jnp.transpose(out_t, (0, 1, 3, 2))


# speedup vs baseline: 1.5884x; 1.0894x over previous
import functools
import math

import jax
import jax.numpy as jnp
from jax import lax
from jax.experimental import pallas as pl
from jax.experimental.pallas import tpu as pltpu
from jax.experimental.pallas import tpu_sc as plsc

D = 512
L = 256
B = 8
VOCAB = 8000
SQRT_D = math.sqrt(float(D))

BM = 256
BN = 8192


def _sc_gather(table, idx):
    ntok = idx.shape[0]
    info = plsc.get_sparse_core_info()
    nw = info.num_cores * info.num_subcores
    b_per_w = ntok // nw

    mesh = plsc.VectorSubcoreMesh(core_axis_name="c", subcore_axis_name="s")

    @functools.partial(
        pl.kernel,
        mesh=mesh,
        out_type=jax.ShapeDtypeStruct((ntok, D), jnp.float32),
        compiler_params=pltpu.CompilerParams(use_tc_tiling_on_sc=True),
        scratch_types=[
            pltpu.VMEM((b_per_w,), jnp.int32),
            pltpu.VMEM((b_per_w, D), jnp.float32),
            pltpu.SemaphoreType.DMA,
        ],
    )
    def gather_kernel(table_hbm, idx_hbm, out_hbm, idx_v, rows_v, sem):
        wid = lax.axis_index("s") * info.num_cores + lax.axis_index("c")
        base = wid * b_per_w
        pltpu.sync_copy(idx_hbm.at[pl.ds(base, b_per_w)], idx_v)
        pltpu.async_copy(table_hbm.at[idx_v], rows_v, sem).wait()
        pltpu.sync_copy(rows_v, out_hbm.at[pl.ds(base, b_per_w)])

    return gather_kernel(table, idx)


def _mm_body(x_ref, pos_ref, w_ref, b_ref, o_ref, xbf):
    n = pl.program_id(0)
    m = pl.program_id(1)

    @pl.when(n == 0)
    def _cache_x():
        ps = pos_ref[pl.ds((m // B) * L, L), :]
        xb = x_ref[pl.ds(m * BM, BM), :] * SQRT_D + ps
        xbf[pl.ds(m * BM, BM), :] = xb.astype(jnp.bfloat16)

    acc = lax.dot_general(w_ref[...], xbf[pl.ds(m * BM, BM), :],
                          (((1,), (1,)), ((), ())),
                          preferred_element_type=jnp.float32)
    o_ref[...] = (acc + b_ref[...]).reshape(1, 1, BN, BM)


def _tc_matmul(x, pos_cat, wbf, bcT):
    ntok = x.shape[0]
    m_blocks = ntok // BM
    n_blocks = (VOCAB + BN - 1) // BN
    return pl.pallas_call(
        _mm_body,
        grid=(n_blocks, m_blocks),
        in_specs=[
            pl.BlockSpec((ntok, D), lambda n, m: (0, 0)),
            pl.BlockSpec((2 * L, D), lambda n, m: (0, 0)),
            pl.BlockSpec((BN, D), lambda n, m: (n, 0)),
            pl.BlockSpec((BN, 1), lambda n, m: (n, 0)),
        ],
        out_specs=pl.BlockSpec((1, 1, BN, BM), lambda n, m: (m // B, m % B, n, 0)),
        out_shape=jax.ShapeDtypeStruct((2, B, VOCAB, BM), jnp.float32),
        scratch_shapes=[
            pltpu.VMEM((ntok, D), jnp.bfloat16),
        ],
    )(x, pos_cat, wbf, bcT)


def kernel(src, tgt, emb, pos_src, pos_tgt, Wc, bc):
    ls = src.shape[1]
    lt = tgt.shape[1]
    idx = jnp.concatenate(
        [src.reshape(-1), tgt.reshape(-1)]).astype(jnp.int32)
    pos_cat = jnp.concatenate([pos_src[:ls, :D], pos_tgt[:lt, :D]], axis=0)
    wbf = Wc.astype(jnp.bfloat16)
    x = _sc_gather(emb, idx)
    out_t = _tc_matmul(x, pos_cat, wbf, bc.reshape(VOCAB, 1))
    return jnp.transpose(out_t, (0, 1, 3, 2))


# speedup vs baseline: 1.6388x; 1.0317x over previous
import functools
import math

import jax
import jax.numpy as jnp
from jax import lax
from jax.experimental import pallas as pl
from jax.experimental.pallas import tpu as pltpu
from jax.experimental.pallas import tpu_sc as plsc

D = 512
L = 256
B = 8
VOCAB = 8000
SQRT_D = math.sqrt(float(D))

BM = 256
BN = 8192


def _sc_gather(table, idx):
    ntok = idx.shape[0]
    info = plsc.get_sparse_core_info()
    nw = info.num_cores * info.num_subcores
    b_per_w = ntok // nw

    mesh = plsc.VectorSubcoreMesh(core_axis_name="c", subcore_axis_name="s")

    @functools.partial(
        pl.kernel,
        mesh=mesh,
        out_type=jax.ShapeDtypeStruct((ntok, D), jnp.float32),
        compiler_params=pltpu.CompilerParams(use_tc_tiling_on_sc=True),
        scratch_types=[
            pltpu.VMEM((b_per_w,), jnp.int32),
            pltpu.VMEM((b_per_w, D), jnp.float32),
            pltpu.SemaphoreType.DMA,
        ],
    )
    def gather_kernel(table_hbm, idx_hbm, out_hbm, idx_v, rows_v, sem):
        wid = lax.axis_index("s") * info.num_cores + lax.axis_index("c")
        base = wid * b_per_w
        pltpu.sync_copy(idx_hbm.at[pl.ds(base, b_per_w)], idx_v)
        pltpu.async_copy(table_hbm.at[idx_v], rows_v, sem).wait()
        pltpu.sync_copy(rows_v, out_hbm.at[pl.ds(base, b_per_w)])

    return gather_kernel(table, idx)


def _mm_body(x_ref, pos_ref, w_ref, b_ref, o_ref, xbf, bcol):
    n = pl.program_id(0)
    m = pl.program_id(1)

    @pl.when(m == 0)
    def _cache_b():
        bcol[...] = jnp.transpose(b_ref[...], (1, 0))

    @pl.when(n == 0)
    def _cache_x():
        ps = pos_ref[pl.ds((m // B) * L, L), :]
        xb = x_ref[pl.ds(m * BM, BM), :] * SQRT_D + ps
        xbf[pl.ds(m * BM, BM), :] = xb.astype(jnp.bfloat16)

    acc = lax.dot_general(w_ref[...], xbf[pl.ds(m * BM, BM), :],
                          (((1,), (1,)), ((), ())),
                          preferred_element_type=jnp.float32)
    o_ref[...] = (acc + bcol[...]).reshape(1, 1, BN, BM)


def _tc_matmul(x, pos_cat, wbf, bcT):
    ntok = x.shape[0]
    m_blocks = ntok // BM
    n_blocks = (VOCAB + BN - 1) // BN
    return pl.pallas_call(
        _mm_body,
        grid=(n_blocks, m_blocks),
        in_specs=[
            pl.BlockSpec((ntok, D), lambda n, m: (0, 0)),
            pl.BlockSpec((2 * L, D), lambda n, m: (0, 0)),
            pl.BlockSpec((BN, D), lambda n, m: (n, 0)),
            pl.BlockSpec((1, BN), lambda n, m: (0, n)),
        ],
        out_specs=pl.BlockSpec((1, 1, BN, BM), lambda n, m: (m // B, m % B, n, 0)),
        out_shape=jax.ShapeDtypeStruct((2, B, VOCAB, BM), jnp.float32),
        scratch_shapes=[
            pltpu.VMEM((ntok, D), jnp.bfloat16),
            pltpu.VMEM(((VOCAB + BN - 1) // BN * BN, 1), jnp.float32),
        ],
    )(x, pos_cat, wbf, bcT)


def kernel(src, tgt, emb, pos_src, pos_tgt, Wc, bc):
    ls = src.shape[1]
    lt = tgt.shape[1]
    idx = jnp.concatenate(
        [src.reshape(-1), tgt.reshape(-1)]).astype(jnp.int32)
    pos_cat = jnp.concatenate([pos_src[:ls, :D], pos_tgt[:lt, :D]], axis=0)
    wbf = Wc.astype(jnp.bfloat16)
    x = _sc_gather(emb, idx)
    out_t = _tc_matmul(x, pos_cat, wbf, bc.reshape(1, VOCAB))
    return jnp.transpose(out_t, (0, 1, 3, 2))
